# baseline (speedup 1.0000x reference)
import jax
import jax.numpy as jnp
from jax import lax
from jax.experimental import pallas as pl
from jax.experimental.pallas import tpu as pltpu
from jax.experimental.pallas import tpu_sc as plsc

NT = 8
MEMD = 128
HIDD = 64
IND = 768
OUTD = 768
NTOK = 4096

_NC = 2
_NS = 16
_NW = _NC * _NS
_CHUNK = NTOK // _NW
_R = 32
_C = NTOK // _R


def _meta_kernel(tv_ref, dst_ref, starts_ref):
    tv = tv_ref[...]
    ii = lax.broadcasted_iota(jnp.int32, (_C, _C), 0)
    jj = lax.broadcasted_iota(jnp.int32, (_C, _C), 1)
    upper = jnp.where(ii <= jj, 1.0, 0.0)
    ri = lax.broadcasted_iota(jnp.int32, (_R, _R), 0)
    rj = lax.broadcasted_iota(jnp.int32, (_R, _R), 1)
    lower = jnp.where(rj < ri, 1.0, 0.0)
    lane16 = lax.broadcasted_iota(jnp.int32, (1, 16), 1)

    dst = jnp.zeros((_R, _C), jnp.float32)
    starts = jnp.zeros((1, 16), jnp.float32)
    g = jnp.float32(0.0)
    for t in range(NT):
        m = jnp.where(tv == t, 1.0, 0.0)
        intra = jnp.dot(m, upper, preferred_element_type=jnp.float32)
        s = intra[:, _C - 1:_C]
        bp = jnp.dot(lower, s, preferred_element_type=jnp.float32)
        rank = intra - m + bp
        dst = dst + m * (rank + g)
        starts = starts + jnp.where(lane16 == t, g, 0.0)
        g = g + bp[_R - 1, 0] + s[_R - 1, 0]
    starts = starts + jnp.where(lane16 >= NT, g, 0.0)
    dst_ref[...] = dst.astype(jnp.int32)
    starts_ref[...] = starts.astype(jnp.int32)


def _scatter_kernel(x_hbm, dst_hbm, xs_hbm, dst_v, xbuf, sem):
    wid = lax.axis_index("s") * _NC + lax.axis_index("c")
    pltpu.sync_copy(dst_hbm.at[pl.ds(wid * _CHUNK, _CHUNK)], dst_v)
    pltpu.sync_copy(x_hbm.at[pl.ds(wid * _CHUNK, _CHUNK)], xbuf)
    pltpu.async_copy(xbuf, xs_hbm.at[dst_v], sem).wait()


_scatter = pl.kernel(
    _scatter_kernel,
    out_type=jax.ShapeDtypeStruct((NTOK, IND), jnp.float32),
    mesh=plsc.VectorSubcoreMesh(core_axis_name="c", subcore_axis_name="s"),
    scratch_types=[
        pltpu.VMEM((_CHUNK,), jnp.int32),
        pltpu.VMEM((_CHUNK, IND), jnp.float32),
        pltpu.SemaphoreType.DMA,
    ],
)


def _gather_kernel(ys_hbm, dst_hbm, out_hbm, dst_v, ybuf, sem):
    wid = lax.axis_index("s") * _NC + lax.axis_index("c")
    pltpu.sync_copy(dst_hbm.at[pl.ds(wid * _CHUNK, _CHUNK)], dst_v)
    pltpu.async_copy(ys_hbm.at[dst_v], ybuf, sem).wait()
    pltpu.sync_copy(ybuf, out_hbm.at[pl.ds(wid * _CHUNK, _CHUNK)])


_gather = pl.kernel(
    _gather_kernel,
    out_type=jax.ShapeDtypeStruct((NTOK, OUTD), jnp.float32),
    mesh=plsc.VectorSubcoreMesh(core_axis_name="c", subcore_axis_name="s"),
    scratch_types=[
        pltpu.VMEM((_CHUNK,), jnp.int32),
        pltpu.VMEM((_CHUNK, OUTD), jnp.float32),
        pltpu.SemaphoreType.DMA,
    ],
)


def _wgen_kernel(m_ref, ww1_ref, wb1_ref, ww2_ref, wb2_ref,
                 bw1_ref, bb1_ref, bw2_ref, bb2_ref, bw3_ref, bb3_ref,
                 w3_ref, b3_ref,
                 wout_ref, ball_ref, hw_ref):
    @pl.when(pl.program_id(0) == 0)
    def _prologue():
        m = m_ref[...]
        h = jnp.dot(m, ww1_ref[...], preferred_element_type=jnp.float32) + wb1_ref[...]
        h = jnp.maximum(h, 0.0)
        h = jnp.dot(h, ww2_ref[...], preferred_element_type=jnp.float32) + wb2_ref[...]
        hw_ref[...] = jnp.maximum(h, 0.0)
        g = jnp.dot(m, bw1_ref[...], preferred_element_type=jnp.float32) + bb1_ref[...]
        g = jnp.maximum(g, 0.0)
        g = jnp.dot(g, bw2_ref[...], preferred_element_type=jnp.float32) + bb2_ref[...]
        g = jnp.maximum(g, 0.0)
        ball_ref[...] = jnp.dot(g, bw3_ref[...], preferred_element_type=jnp.float32) + bb3_ref[...]

    w2 = (jnp.dot(hw_ref[...], w3_ref[...], preferred_element_type=jnp.float32)
          + b3_ref[...])
    wout_ref[...] = w2.reshape(wout_ref.shape).astype(jnp.bfloat16)


BN = 512


def _apply_kernel(s_ref, x_ref, w_ref, b_ref, out_ref):
    row0 = pl.program_id(0) * BN
    xb = x_ref[...].astype(jnp.bfloat16)
    riota = lax.broadcasted_iota(jnp.int32, (BN, 1), 0)
    out_ref[...] = jnp.zeros(out_ref.shape, jnp.float32)
    for t in range(NT):
        lo = s_ref[t] - row0
        hi = s_ref[t + 1] - row0

        @pl.when((lo < BN) & (hi > 0) & (hi > lo))
        def _seg(t=t, lo=lo, hi=hi):
            m = (riota >= lo) & (riota < hi)
            xt = jnp.where(m, xb, jnp.bfloat16(0.0))
            out_ref[...] += (jnp.dot(xt, w_ref[t], preferred_element_type=jnp.float32)
                             + jnp.where(m, b_ref[t:t + 1, :], 0.0))


def kernel(x, type_vec, edge_feas_dict,
           wg_w1, wg_b1, wg_w2, wg_b2, wg_w3, wg_b3,
           bg_w1, bg_b1, bg_w2, bg_b2, bg_w3, bg_b3):
    tv = type_vec.astype(jnp.int32)

    CB = 98304
    ncb = (IND * OUTD) // CB
    const = lambda shape: pl.BlockSpec(shape, lambda j: tuple(0 for _ in shape))
    w_all, ball = pl.pallas_call(
        _wgen_kernel,
        grid=(ncb,),
        in_specs=[
            const((NT, MEMD)),
            const((MEMD, HIDD)), const((1, HIDD)),
            const((HIDD, HIDD)), const((1, HIDD)),
            const((MEMD, HIDD)), const((1, HIDD)),
            const((HIDD, HIDD)), const((1, HIDD)),
            const((HIDD, OUTD)), const((1, OUTD)),
            pl.BlockSpec((HIDD, CB), lambda j: (0, j)),
            pl.BlockSpec((1, CB), lambda j: (0, j)),
        ],
        out_specs=(pl.BlockSpec((NT, CB // OUTD, OUTD), lambda j: (0, j, 0)),
                   const((NT, OUTD))),
        out_shape=(jax.ShapeDtypeStruct((NT, IND, OUTD), jnp.bfloat16),
                   jax.ShapeDtypeStruct((NT, OUTD), jnp.float32)),
        scratch_shapes=[pltpu.VMEM((NT, HIDD), jnp.float32)],
    )(edge_feas_dict,
      wg_w1, wg_b1.reshape(1, HIDD), wg_w2, wg_b2.reshape(1, HIDD),
      bg_w1, bg_b1.reshape(1, HIDD), bg_w2, bg_b2.reshape(1, HIDD),
      bg_w3, bg_b3.reshape(1, OUTD),
      wg_w3, wg_b3.reshape(1, IND * OUTD))

    dst2, starts = pl.pallas_call(
        _meta_kernel,
        out_shape=(jax.ShapeDtypeStruct((_R, _C), jnp.int32),
                   jax.ShapeDtypeStruct((1, 16), jnp.int32)),
    )(tv.reshape(_R, _C))
    dst = dst2.reshape(NTOK)

    xs = _scatter(x, dst)

    ys = pl.pallas_call(
        _apply_kernel,
        grid_spec=pltpu.PrefetchScalarGridSpec(
            num_scalar_prefetch=1,
            grid=(NTOK // BN,),
            in_specs=[
                pl.BlockSpec((BN, IND), lambda n, s: (n, 0)),
                pl.BlockSpec((NT, IND, OUTD), lambda n, s: (0, 0, 0)),
                pl.BlockSpec((NT, OUTD), lambda n, s: (0, 0)),
            ],
            out_specs=pl.BlockSpec((BN, OUTD), lambda n, s: (n, 0)),
        ),
        out_shape=jax.ShapeDtypeStruct((NTOK, OUTD), jnp.float32),
    )(starts.reshape(16), xs, w_all, ball)

    return _gather(ys, dst)


# speedup vs baseline: 1.0031x; 1.0031x over previous
import jax
import jax.numpy as jnp
from jax import lax
from jax.experimental import pallas as pl
from jax.experimental.pallas import tpu as pltpu
from jax.experimental.pallas import tpu_sc as plsc

NT = 8
MEMD = 128
HIDD = 64
IND = 768
OUTD = 768
NTOK = 4096

_NC = 2
_NS = 16
_NW = _NC * _NS
_CHUNK = NTOK // _NW
_R = 32
_C = NTOK // _R


def _meta_kernel(tv_ref, dst_ref, starts_ref):
    tv = tv_ref[...]
    ii = lax.broadcasted_iota(jnp.int32, (_C, _C), 0)
    jj = lax.broadcasted_iota(jnp.int32, (_C, _C), 1)
    upper = jnp.where(ii <= jj, 1.0, 0.0)
    ri = lax.broadcasted_iota(jnp.int32, (_R, _R), 0)
    rj = lax.broadcasted_iota(jnp.int32, (_R, _R), 1)
    lower = jnp.where(rj < ri, 1.0, 0.0)
    lane16 = lax.broadcasted_iota(jnp.int32, (1, 16), 1)

    dst = jnp.zeros((_R, _C), jnp.float32)
    starts = jnp.zeros((1, 16), jnp.float32)
    g = jnp.float32(0.0)
    for t in range(NT):
        m = jnp.where(tv == t, 1.0, 0.0)
        intra = jnp.dot(m, upper, preferred_element_type=jnp.float32)
        s = intra[:, _C - 1:_C]
        bp = jnp.dot(lower, s, preferred_element_type=jnp.float32)
        rank = intra - m + bp
        dst = dst + m * (rank + g)
        starts = starts + jnp.where(lane16 == t, g, 0.0)
        g = g + bp[_R - 1, 0] + s[_R - 1, 0]
    starts = starts + jnp.where(lane16 >= NT, g, 0.0)
    dst_ref[...] = dst.astype(jnp.int32)
    starts_ref[...] = starts.astype(jnp.int32)


def _scatter_kernel(x_hbm, dst_hbm, xs_hbm, dst_v, xbuf, sem):
    wid = lax.axis_index("s") * _NC + lax.axis_index("c")
    pltpu.sync_copy(dst_hbm.at[pl.ds(wid * _CHUNK, _CHUNK)], dst_v)
    pltpu.sync_copy(x_hbm.at[pl.ds(wid * _CHUNK, _CHUNK)], xbuf)
    pltpu.async_copy(xbuf, xs_hbm.at[dst_v], sem).wait()


_scatter = pl.kernel(
    _scatter_kernel,
    out_type=jax.ShapeDtypeStruct((NTOK, IND), jnp.float32),
    mesh=plsc.VectorSubcoreMesh(core_axis_name="c", subcore_axis_name="s"),
    scratch_types=[
        pltpu.VMEM((_CHUNK,), jnp.int32),
        pltpu.VMEM((_CHUNK, IND), jnp.float32),
        pltpu.SemaphoreType.DMA,
    ],
)


def _gather_kernel(ys_hbm, dst_hbm, out_hbm, dst_v, ybuf, sem):
    wid = lax.axis_index("s") * _NC + lax.axis_index("c")
    pltpu.sync_copy(dst_hbm.at[pl.ds(wid * _CHUNK, _CHUNK)], dst_v)
    pltpu.async_copy(ys_hbm.at[dst_v], ybuf, sem).wait()
    pltpu.sync_copy(ybuf, out_hbm.at[pl.ds(wid * _CHUNK, _CHUNK)])


_gather = pl.kernel(
    _gather_kernel,
    out_type=jax.ShapeDtypeStruct((NTOK, OUTD), jnp.float32),
    mesh=plsc.VectorSubcoreMesh(core_axis_name="c", subcore_axis_name="s"),
    scratch_types=[
        pltpu.VMEM((_CHUNK,), jnp.int32),
        pltpu.VMEM((_CHUNK, OUTD), jnp.float32),
        pltpu.SemaphoreType.DMA,
    ],
)


def _wgen_kernel(m_ref, ww1_ref, wb1_ref, ww2_ref, wb2_ref,
                 bw1_ref, bb1_ref, bw2_ref, bb2_ref, bw3_ref, bb3_ref,
                 w3_ref, b3_ref,
                 wout_ref, ball_ref, hw_ref):
    @pl.when(pl.program_id(0) == 0)
    def _prologue():
        m = m_ref[...]
        h = jnp.dot(m, ww1_ref[...], preferred_element_type=jnp.float32) + wb1_ref[...]
        h = jnp.maximum(h, 0.0)
        h = jnp.dot(h, ww2_ref[...], preferred_element_type=jnp.float32) + wb2_ref[...]
        hw_ref[...] = jnp.maximum(h, 0.0)
        g = jnp.dot(m, bw1_ref[...], preferred_element_type=jnp.float32) + bb1_ref[...]
        g = jnp.maximum(g, 0.0)
        g = jnp.dot(g, bw2_ref[...], preferred_element_type=jnp.float32) + bb2_ref[...]
        g = jnp.maximum(g, 0.0)
        ball_ref[...] = jnp.dot(g, bw3_ref[...], preferred_element_type=jnp.float32) + bb3_ref[...]

    w2 = (jnp.dot(hw_ref[...], w3_ref[...], preferred_element_type=jnp.float32)
          + b3_ref[...])
    wout_ref[...] = w2.reshape(wout_ref.shape).astype(jnp.bfloat16)


BN = 512


def _apply_kernel(s_ref, x_ref, w_ref, b_ref, out_ref):
    row0 = pl.program_id(0) * BN
    row_last = row0 + BN - 1
    xb = x_ref[...].astype(jnp.bfloat16)
    riota = lax.broadcasted_iota(jnp.int32, (BN, 1), 0)
    tlo = jnp.int32(0)
    thi = jnp.int32(0)
    for t in range(1, NT):
        tlo = tlo + jnp.where(s_ref[t] <= row0, 1, 0)
        thi = thi + jnp.where(s_ref[t] <= row_last, 1, 0)
    out_ref[...] = jnp.zeros(out_ref.shape, jnp.float32)

    def _seg(t, carry):
        lo = s_ref[t] - row0
        hi = s_ref[t + 1] - row0
        m = (riota >= lo) & (riota < hi)
        xt = jnp.where(m, xb, jnp.bfloat16(0.0))
        w_t = w_ref[t]
        b_t = b_ref[pl.ds(t, 1), :]
        out_ref[...] += (jnp.dot(xt, w_t, preferred_element_type=jnp.float32)
                         + jnp.where(m, b_t, 0.0))
        return carry

    lax.fori_loop(tlo, thi + 1, _seg, 0)


def kernel(x, type_vec, edge_feas_dict,
           wg_w1, wg_b1, wg_w2, wg_b2, wg_w3, wg_b3,
           bg_w1, bg_b1, bg_w2, bg_b2, bg_w3, bg_b3):
    tv = type_vec.astype(jnp.int32)

    CB = 98304
    ncb = (IND * OUTD) // CB
    const = lambda shape: pl.BlockSpec(shape, lambda j: tuple(0 for _ in shape))
    w_all, ball = pl.pallas_call(
        _wgen_kernel,
        grid=(ncb,),
        in_specs=[
            const((NT, MEMD)),
            const((MEMD, HIDD)), const((1, HIDD)),
            const((HIDD, HIDD)), const((1, HIDD)),
            const((MEMD, HIDD)), const((1, HIDD)),
            const((HIDD, HIDD)), const((1, HIDD)),
            const((HIDD, OUTD)), const((1, OUTD)),
            pl.BlockSpec((HIDD, CB), lambda j: (0, j)),
            pl.BlockSpec((1, CB), lambda j: (0, j)),
        ],
        out_specs=(pl.BlockSpec((NT, CB // OUTD, OUTD), lambda j: (0, j, 0)),
                   const((NT, OUTD))),
        out_shape=(jax.ShapeDtypeStruct((NT, IND, OUTD), jnp.bfloat16),
                   jax.ShapeDtypeStruct((NT, OUTD), jnp.float32)),
        scratch_shapes=[pltpu.VMEM((NT, HIDD), jnp.float32)],
    )(edge_feas_dict,
      wg_w1, wg_b1.reshape(1, HIDD), wg_w2, wg_b2.reshape(1, HIDD),
      bg_w1, bg_b1.reshape(1, HIDD), bg_w2, bg_b2.reshape(1, HIDD),
      bg_w3, bg_b3.reshape(1, OUTD),
      wg_w3, wg_b3.reshape(1, IND * OUTD))

    dst2, starts = pl.pallas_call(
        _meta_kernel,
        out_shape=(jax.ShapeDtypeStruct((_R, _C), jnp.int32),
                   jax.ShapeDtypeStruct((1, 16), jnp.int32)),
    )(tv.reshape(_R, _C))
    dst = dst2.reshape(NTOK)

    xs = _scatter(x, dst)

    ys = pl.pallas_call(
        _apply_kernel,
        grid_spec=pltpu.PrefetchScalarGridSpec(
            num_scalar_prefetch=1,
            grid=(NTOK // BN,),
            in_specs=[
                pl.BlockSpec((BN, IND), lambda n, s: (n, 0)),
                pl.BlockSpec((NT, IND, OUTD), lambda n, s: (0, 0, 0)),
                pl.BlockSpec((NT, OUTD), lambda n, s: (0, 0)),
            ],
            out_specs=pl.BlockSpec((BN, OUTD), lambda n, s: (n, 0)),
        ),
        out_shape=jax.ShapeDtypeStruct((NTOK, OUTD), jnp.float32),
    )(starts.reshape(16), xs, w_all, ball)

    return _gather(ys, dst)


# speedup vs baseline: 1.0327x; 1.0296x over previous
import jax
import jax.numpy as jnp
from jax import lax
from jax.experimental import pallas as pl
from jax.experimental.pallas import tpu as pltpu
from jax.experimental.pallas import tpu_sc as plsc

NT = 8
MEMD = 128
HIDD = 64
IND = 768
OUTD = 768
NTOK = 4096

_NC = 2
_NS = 16
_NW = _NC * _NS
_CHUNK = NTOK // _NW
_R = 32
_C = NTOK // _R


def _meta_kernel(tv_ref, dst_ref, starts_ref):
    tv = tv_ref[...]
    ii = lax.broadcasted_iota(jnp.int32, (_C, _C), 0)
    jj = lax.broadcasted_iota(jnp.int32, (_C, _C), 1)
    upper = jnp.where(ii <= jj, 1.0, 0.0)
    ri = lax.broadcasted_iota(jnp.int32, (_R, _R), 0)
    rj = lax.broadcasted_iota(jnp.int32, (_R, _R), 1)
    lower = jnp.where(rj < ri, 1.0, 0.0)
    lane16 = lax.broadcasted_iota(jnp.int32, (1, 16), 1)

    dst = jnp.zeros((_R, _C), jnp.float32)
    starts = jnp.zeros((1, 16), jnp.float32)
    g = jnp.float32(0.0)
    for t in range(NT):
        m = jnp.where(tv == t, 1.0, 0.0)
        intra = jnp.dot(m, upper, preferred_element_type=jnp.float32)
        s = intra[:, _C - 1:_C]
        bp = jnp.dot(lower, s, preferred_element_type=jnp.float32)
        rank = intra - m + bp
        dst = dst + m * (rank + g)
        starts = starts + jnp.where(lane16 == t, g, 0.0)
        g = g + bp[_R - 1, 0] + s[_R - 1, 0]
    starts = starts + jnp.where(lane16 >= NT, g, 0.0)
    dst_ref[...] = dst.astype(jnp.int32)
    starts_ref[...] = starts.astype(jnp.int32)


def _scatter_kernel(x_hbm, dst_hbm, xs_hbm, dst_v, xbuf, sem):
    wid = lax.axis_index("s") * _NC + lax.axis_index("c")
    pltpu.sync_copy(dst_hbm.at[pl.ds(wid * _CHUNK, _CHUNK)], dst_v)
    pltpu.sync_copy(x_hbm.at[pl.ds(wid * _CHUNK, _CHUNK)], xbuf)
    pltpu.async_copy(xbuf, xs_hbm.at[dst_v], sem).wait()


_scatter = pl.kernel(
    _scatter_kernel,
    out_type=jax.ShapeDtypeStruct((NTOK, IND), jnp.float32),
    mesh=plsc.VectorSubcoreMesh(core_axis_name="c", subcore_axis_name="s"),
    scratch_types=[
        pltpu.VMEM((_CHUNK,), jnp.int32),
        pltpu.VMEM((_CHUNK, IND), jnp.float32),
        pltpu.SemaphoreType.DMA,
    ],
)


def _gather_kernel(ys_hbm, dst_hbm, out_hbm, dst_v, ybuf, sem):
    wid = lax.axis_index("s") * _NC + lax.axis_index("c")
    pltpu.sync_copy(dst_hbm.at[pl.ds(wid * _CHUNK, _CHUNK)], dst_v)
    pltpu.async_copy(ys_hbm.at[dst_v], ybuf, sem).wait()
    pltpu.sync_copy(ybuf, out_hbm.at[pl.ds(wid * _CHUNK, _CHUNK)])


_gather = pl.kernel(
    _gather_kernel,
    out_type=jax.ShapeDtypeStruct((NTOK, OUTD), jnp.float32),
    mesh=plsc.VectorSubcoreMesh(core_axis_name="c", subcore_axis_name="s"),
    scratch_types=[
        pltpu.VMEM((_CHUNK,), jnp.int32),
        pltpu.VMEM((_CHUNK, OUTD), jnp.float32),
        pltpu.SemaphoreType.DMA,
    ],
)


def _wgen_kernel(m_ref, ww1_ref, wb1_ref, ww2_ref, wb2_ref,
                 bw1_ref, bb1_ref, bw2_ref, bb2_ref, bw3_ref, bb3_ref,
                 w3a_ref, w3b_ref, b3a_ref, b3b_ref,
                 wout_ref, ball_ref, hw_ref):
    @pl.when(pl.program_id(0) == 0)
    def _prologue():
        m = m_ref[...]
        h = jnp.dot(m, ww1_ref[...], preferred_element_type=jnp.float32) + wb1_ref[...]
        h = jnp.maximum(h, 0.0)
        h = jnp.dot(h, ww2_ref[...], preferred_element_type=jnp.float32) + wb2_ref[...]
        hw_ref[...] = jnp.maximum(h, 0.0)
        g = jnp.dot(m, bw1_ref[...], preferred_element_type=jnp.float32) + bb1_ref[...]
        g = jnp.maximum(g, 0.0)
        g = jnp.dot(g, bw2_ref[...], preferred_element_type=jnp.float32) + bb2_ref[...]
        g = jnp.maximum(g, 0.0)
        ball_ref[...] = jnp.dot(g, bw3_ref[...], preferred_element_type=jnp.float32) + bb3_ref[...]

    half = wout_ref.shape[1] // 2
    w2a = (jnp.dot(hw_ref[...], w3a_ref[...], preferred_element_type=jnp.float32)
           + b3a_ref[...])
    wout_ref[:, :half, :] = w2a.reshape(NT, half, OUTD).astype(jnp.bfloat16)
    w2b = (jnp.dot(hw_ref[...], w3b_ref[...], preferred_element_type=jnp.float32)
           + b3b_ref[...])
    wout_ref[:, half:, :] = w2b.reshape(NT, half, OUTD).astype(jnp.bfloat16)


BN = 512


def _apply_kernel(s_ref, x_ref, w_ref, b_ref, out_ref):
    row0 = pl.program_id(0) * BN
    row_last = row0 + BN - 1
    xb = x_ref[...].astype(jnp.bfloat16)
    riota = lax.broadcasted_iota(jnp.int32, (BN, 1), 0)
    tlo = jnp.int32(0)
    thi = jnp.int32(0)
    for t in range(1, NT):
        tlo = tlo + jnp.where(s_ref[t] <= row0, 1, 0)
        thi = thi + jnp.where(s_ref[t] <= row_last, 1, 0)
    out_ref[...] = jnp.zeros(out_ref.shape, jnp.float32)

    def _seg(t, carry):
        lo = s_ref[t] - row0
        hi = s_ref[t + 1] - row0
        m = (riota >= lo) & (riota < hi)
        xt = jnp.where(m, xb, jnp.bfloat16(0.0))
        w_t = w_ref[t]
        b_t = b_ref[pl.ds(t, 1), :]
        out_ref[...] += (jnp.dot(xt, w_t, preferred_element_type=jnp.float32)
                         + jnp.where(m, b_t, 0.0))
        return carry

    lax.fori_loop(tlo, thi + 1, _seg, 0)


def kernel(x, type_vec, edge_feas_dict,
           wg_w1, wg_b1, wg_w2, wg_b2, wg_w3, wg_b3,
           bg_w1, bg_b1, bg_w2, bg_b2, bg_w3, bg_b3):
    tv = type_vec.astype(jnp.int32)

    CB = 49152
    ncb = (IND * OUTD) // (2 * CB)
    const = lambda shape: pl.BlockSpec(shape, lambda j: tuple(0 for _ in shape))
    w3b_shaped = wg_b3.reshape(1, IND * OUTD)
    w_all, ball = pl.pallas_call(
        _wgen_kernel,
        grid=(ncb,),
        in_specs=[
            const((NT, MEMD)),
            const((MEMD, HIDD)), const((1, HIDD)),
            const((HIDD, HIDD)), const((1, HIDD)),
            const((MEMD, HIDD)), const((1, HIDD)),
            const((HIDD, HIDD)), const((1, HIDD)),
            const((HIDD, OUTD)), const((1, OUTD)),
            pl.BlockSpec((HIDD, CB), lambda j: (0, 2 * j)),
            pl.BlockSpec((HIDD, CB), lambda j: (0, 2 * j + 1)),
            pl.BlockSpec((1, CB), lambda j: (0, 2 * j)),
            pl.BlockSpec((1, CB), lambda j: (0, 2 * j + 1)),
        ],
        out_specs=(pl.BlockSpec((NT, 2 * CB // OUTD, OUTD), lambda j: (0, j, 0)),
                   const((NT, OUTD))),
        out_shape=(jax.ShapeDtypeStruct((NT, IND, OUTD), jnp.bfloat16),
                   jax.ShapeDtypeStruct((NT, OUTD), jnp.float32)),
        scratch_shapes=[pltpu.VMEM((NT, HIDD), jnp.float32)],
    )(edge_feas_dict,
      wg_w1, wg_b1.reshape(1, HIDD), wg_w2, wg_b2.reshape(1, HIDD),
      bg_w1, bg_b1.reshape(1, HIDD), bg_w2, bg_b2.reshape(1, HIDD),
      bg_w3, bg_b3.reshape(1, OUTD),
      wg_w3, wg_w3, w3b_shaped, w3b_shaped)

    dst2, starts = pl.pallas_call(
        _meta_kernel,
        out_shape=(jax.ShapeDtypeStruct((_R, _C), jnp.int32),
                   jax.ShapeDtypeStruct((1, 16), jnp.int32)),
    )(tv.reshape(_R, _C))
    dst = dst2.reshape(NTOK)

    xs = _scatter(x, dst)

    ys = pl.pallas_call(
        _apply_kernel,
        grid_spec=pltpu.PrefetchScalarGridSpec(
            num_scalar_prefetch=1,
            grid=(NTOK // BN,),
            in_specs=[
                pl.BlockSpec((BN, IND), lambda n, s: (n, 0)),
                pl.BlockSpec((NT, IND, OUTD), lambda n, s: (0, 0, 0)),
                pl.BlockSpec((NT, OUTD), lambda n, s: (0, 0)),
            ],
            out_specs=pl.BlockSpec((BN, OUTD), lambda n, s: (n, 0)),
        ),
        out_shape=jax.ShapeDtypeStruct((NTOK, OUTD), jnp.float32),
    )(starts.reshape(16), xs, w_all, ball)

    return _gather(ys, dst)


# speedup vs baseline: 1.0358x; 1.0030x over previous
import jax
import jax.numpy as jnp
from jax import lax
from jax.experimental import pallas as pl
from jax.experimental.pallas import tpu as pltpu
from jax.experimental.pallas import tpu_sc as plsc

NT = 8
MEMD = 128
HIDD = 64
IND = 768
OUTD = 768
NTOK = 4096

_NC = 2
_NS = 16
_NW = _NC * _NS
_CHUNK = NTOK // _NW
_R = 32
_C = NTOK // _R
_NST = 4
_CBW = 18432


def _meta_kernel(tv_ref, dst_ref, starts_ref):
    tv = tv_ref[...]
    ii = lax.broadcasted_iota(jnp.int32, (_C, _C), 0)
    jj = lax.broadcasted_iota(jnp.int32, (_C, _C), 1)
    upper = jnp.where(ii <= jj, 1.0, 0.0)
    ri = lax.broadcasted_iota(jnp.int32, (_R, _R), 0)
    rj = lax.broadcasted_iota(jnp.int32, (_R, _R), 1)
    lower = jnp.where(rj < ri, 1.0, 0.0)
    lane16 = lax.broadcasted_iota(jnp.int32, (1, 16), 1)

    dst = jnp.zeros((_R, _C), jnp.float32)
    starts = jnp.zeros((1, 16), jnp.float32)
    g = jnp.float32(0.0)
    for t in range(NT):
        m = jnp.where(tv == t, 1.0, 0.0)
        intra = jnp.dot(m, upper, preferred_element_type=jnp.float32)
        s = intra[:, _C - 1:_C]
        bp = jnp.dot(lower, s, preferred_element_type=jnp.float32)
        rank = intra - m + bp
        dst = dst + m * (rank + g)
        starts = starts + jnp.where(lane16 == t, g, 0.0)
        g = g + bp[_R - 1, 0] + s[_R - 1, 0]
    starts = starts + jnp.where(lane16 >= NT, g, 0.0)
    dst_ref[...] = dst.astype(jnp.int32)
    starts_ref[...] = starts.astype(jnp.int32)


def _scatter_kernel(x_hbm, dst_hbm, xs_hbm, dst_v, xbuf, sem):
    wid = lax.axis_index("s") * _NC + lax.axis_index("c")
    pltpu.sync_copy(dst_hbm.at[pl.ds(wid * _CHUNK, _CHUNK)], dst_v)
    pltpu.sync_copy(x_hbm.at[pl.ds(wid * _CHUNK, _CHUNK)], xbuf)
    pltpu.async_copy(xbuf, xs_hbm.at[dst_v], sem).wait()


_scatter = pl.kernel(
    _scatter_kernel,
    out_type=jax.ShapeDtypeStruct((NTOK, IND), jnp.float32),
    mesh=plsc.VectorSubcoreMesh(core_axis_name="c", subcore_axis_name="s"),
    scratch_types=[
        pltpu.VMEM((_CHUNK,), jnp.int32),
        pltpu.VMEM((_CHUNK, IND), jnp.float32),
        pltpu.SemaphoreType.DMA,
    ],
)


def _gather_kernel(ys_hbm, dst_hbm, out_hbm, dst_v, ybuf, sem):
    wid = lax.axis_index("s") * _NC + lax.axis_index("c")
    pltpu.sync_copy(dst_hbm.at[pl.ds(wid * _CHUNK, _CHUNK)], dst_v)
    pltpu.async_copy(ys_hbm.at[dst_v], ybuf, sem).wait()
    pltpu.sync_copy(ybuf, out_hbm.at[pl.ds(wid * _CHUNK, _CHUNK)])


_gather = pl.kernel(
    _gather_kernel,
    out_type=jax.ShapeDtypeStruct((NTOK, OUTD), jnp.float32),
    mesh=plsc.VectorSubcoreMesh(core_axis_name="c", subcore_axis_name="s"),
    scratch_types=[
        pltpu.VMEM((_CHUNK,), jnp.int32),
        pltpu.VMEM((_CHUNK, OUTD), jnp.float32),
        pltpu.SemaphoreType.DMA,
    ],
)


def _wgen_kernel(m_ref, ww1_ref, wb1_ref, ww2_ref, wb2_ref,
                 bw1_ref, bb1_ref, bw2_ref, bb2_ref, bw3_ref, bb3_ref,
                 *rest_refs):
    w3_refs = rest_refs[:_NST]
    b3_refs = rest_refs[_NST:2 * _NST]
    wout_ref, ball_ref, hw_ref = rest_refs[2 * _NST:]

    @pl.when(pl.program_id(0) == 0)
    def _prologue():
        m = m_ref[...]
        h = jnp.dot(m, ww1_ref[...], preferred_element_type=jnp.float32) + wb1_ref[...]
        h = jnp.maximum(h, 0.0)
        h = jnp.dot(h, ww2_ref[...], preferred_element_type=jnp.float32) + wb2_ref[...]
        hw_ref[...] = jnp.maximum(h, 0.0)
        g = jnp.dot(m, bw1_ref[...], preferred_element_type=jnp.float32) + bb1_ref[...]
        g = jnp.maximum(g, 0.0)
        g = jnp.dot(g, bw2_ref[...], preferred_element_type=jnp.float32) + bb2_ref[...]
        g = jnp.maximum(g, 0.0)
        ball_ref[...] = jnp.dot(g, bw3_ref[...], preferred_element_type=jnp.float32) + bb3_ref[...]

    rows = wout_ref.shape[1] // _NST
    for i in range(_NST):
        w2 = (jnp.dot(hw_ref[...], w3_refs[i][...],
                      preferred_element_type=jnp.float32) + b3_refs[i][...])
        wout_ref[:, i * rows:(i + 1) * rows, :] = (
            w2.reshape(NT, rows, OUTD).astype(jnp.bfloat16))


BN = 512


def _apply_kernel(s_ref, x_ref, w_ref, b_ref, out_ref):
    row0 = pl.program_id(0) * BN
    row_last = row0 + BN - 1
    xb = x_ref[...].astype(jnp.bfloat16)
    riota = lax.broadcasted_iota(jnp.int32, (BN, 1), 0)
    tlo = jnp.int32(0)
    thi = jnp.int32(0)
    for t in range(1, NT):
        tlo = tlo + jnp.where(s_ref[t] <= row0, 1, 0)
        thi = thi + jnp.where(s_ref[t] <= row_last, 1, 0)
    out_ref[...] = jnp.zeros(out_ref.shape, jnp.float32)

    def _seg(t, carry):
        lo = s_ref[t] - row0
        hi = s_ref[t + 1] - row0
        m = (riota >= lo) & (riota < hi)
        xt = jnp.where(m, xb, jnp.bfloat16(0.0))
        w_t = w_ref[t]
        b_t = b_ref[pl.ds(t, 1), :]
        out_ref[...] += (jnp.dot(xt, w_t, preferred_element_type=jnp.float32)
                         + jnp.where(m, b_t, 0.0))
        return carry

    lax.fori_loop(tlo, thi + 1, _seg, 0)


def kernel(x, type_vec, edge_feas_dict,
           wg_w1, wg_b1, wg_w2, wg_b2, wg_w3, wg_b3,
           bg_w1, bg_b1, bg_w2, bg_b2, bg_w3, bg_b3):
    tv = type_vec.astype(jnp.int32)

    ncb = (IND * OUTD) // (_NST * _CBW)
    const = lambda shape: pl.BlockSpec(shape, lambda j: tuple(0 for _ in shape))
    w3b_shaped = wg_b3.reshape(1, IND * OUTD)

    def _w3_spec(i):
        return pl.BlockSpec((HIDD, _CBW), lambda j, i=i: (0, _NST * j + i))

    def _b3_spec(i):
        return pl.BlockSpec((1, _CBW), lambda j, i=i: (0, _NST * j + i))

    w_all, ball = pl.pallas_call(
        _wgen_kernel,
        grid=(ncb,),
        in_specs=[
            const((NT, MEMD)),
            const((MEMD, HIDD)), const((1, HIDD)),
            const((HIDD, HIDD)), const((1, HIDD)),
            const((MEMD, HIDD)), const((1, HIDD)),
            const((HIDD, HIDD)), const((1, HIDD)),
            const((HIDD, OUTD)), const((1, OUTD)),
        ] + [_w3_spec(i) for i in range(_NST)] + [_b3_spec(i) for i in range(_NST)],
        out_specs=(pl.BlockSpec((NT, _NST * _CBW // OUTD, OUTD), lambda j: (0, j, 0)),
                   const((NT, OUTD))),
        out_shape=(jax.ShapeDtypeStruct((NT, IND, OUTD), jnp.bfloat16),
                   jax.ShapeDtypeStruct((NT, OUTD), jnp.float32)),
        scratch_shapes=[pltpu.VMEM((NT, HIDD), jnp.float32)],
    )(edge_feas_dict,
      wg_w1, wg_b1.reshape(1, HIDD), wg_w2, wg_b2.reshape(1, HIDD),
      bg_w1, bg_b1.reshape(1, HIDD), bg_w2, bg_b2.reshape(1, HIDD),
      bg_w3, bg_b3.reshape(1, OUTD),
      *([wg_w3] * _NST), *([w3b_shaped] * _NST))

    dst2, starts = pl.pallas_call(
        _meta_kernel,
        out_shape=(jax.ShapeDtypeStruct((_R, _C), jnp.int32),
                   jax.ShapeDtypeStruct((1, 16), jnp.int32)),
    )(tv.reshape(_R, _C))
    dst = dst2.reshape(NTOK)

    xs = _scatter(x, dst)

    ys = pl.pallas_call(
        _apply_kernel,
        grid_spec=pltpu.PrefetchScalarGridSpec(
            num_scalar_prefetch=1,
            grid=(NTOK // BN,),
            in_specs=[
                pl.BlockSpec((BN, IND), lambda n, s: (n, 0)),
                pl.BlockSpec((NT, IND, OUTD), lambda n, s: (0, 0, 0)),
                pl.BlockSpec((NT, OUTD), lambda n, s: (0, 0)),
            ],
            out_specs=pl.BlockSpec((BN, OUTD), lambda n, s: (n, 0)),
        ),
        out_shape=jax.ShapeDtypeStruct((NTOK, OUTD), jnp.float32),
    )(starts.reshape(16), xs, w_all, ball)

    return _gather(ys, dst)
